# baseline (speedup 1.0000x reference)
import functools

import jax
import jax.numpy as jnp
from jax import lax
from jax.experimental import pallas as pl
from jax.experimental.pallas import tpu as pltpu
from jax.experimental.pallas import tpu_sc as plsc

NC = 2
NS = 16
L = 16
C = 80
C1 = 40
RI = 16


def _sc_aggregate(nf_split, ef_r, src, dst, N, E, DH, DE):
    per_tile1 = E // NS
    n1 = per_tile1 // C1
    per_tile2 = E // (NC * NS)
    n2 = per_tile2 // C
    assert per_tile1 % C1 == 0 and per_tile2 % C == 0 and N % RI == 0
    K = 5
    NB1, NB2 = n1 // K, n2 // K
    assert n1 % K == 0 and n2 % K == 0
    assert NB1 % 2 == 0 and NB1 >= 6
    assert NB2 % 2 == 1 and NB2 >= 3
    KI = N // RI
    KI_PT = (KI + NS - 1) // NS

    mesh = plsc.VectorSubcoreMesh(core_axis_name="c", subcore_axis_name="s")

    @functools.partial(
        pl.kernel,
        out_type=(
            jax.ShapeDtypeStruct((NC, N, DH), jnp.float32),
            jax.ShapeDtypeStruct((NC, N, 2 * DE), jnp.float32),
        ),
        mesh=mesh,
        compiler_params=pltpu.CompilerParams(use_tc_tiling_on_sc=False),
        scratch_types=[
            pltpu.VMEM_SHARED((N, DH), jnp.float32),
            pltpu.VMEM_SHARED((N, DE), jnp.float32),
            pltpu.VMEM_SHARED((N, DE), jnp.float32),
            pltpu.VMEM((K, 1, C1), jnp.int32),
            pltpu.VMEM((K, 1, C1), jnp.int32),
            pltpu.VMEM((K, 1, C1), jnp.int32),
            pltpu.VMEM((K, 1, C1), jnp.int32),
            pltpu.VMEM((K, 1, C1), jnp.int32),
            pltpu.VMEM((K, 1, C1), jnp.int32),
            pltpu.VMEM((K, 1, C), jnp.int32),
            pltpu.VMEM((K, 1, C), jnp.int32),
            pltpu.VMEM((K, 1, C), jnp.int32),
            pltpu.VMEM((K, 1, C), jnp.int32),
            pltpu.VMEM((K, C1, DH), jnp.float32),
            pltpu.VMEM((K, C1, DH), jnp.float32),
            pltpu.VMEM((K, C * DE // 128, 128), jnp.float32),
            pltpu.VMEM((K, C * DE // 128, 128), jnp.float32),
            pltpu.VMEM((K, C, DE), jnp.float32),
            pltpu.VMEM((K, C, DE), jnp.float32),
            pltpu.VMEM((C, DE), jnp.float32),
            pltpu.VMEM((RI, DH), jnp.float32),
            pltpu.VMEM((RI, DE), jnp.float32),
            pltpu.SemaphoreType.DMA,
            pltpu.SemaphoreType.DMA,
            pltpu.SemaphoreType.DMA,
            pltpu.SemaphoreType.DMA,
            pltpu.SemaphoreType.DMA,
            pltpu.SemaphoreType.DMA,
            pltpu.SemaphoreType.DMA,
            pltpu.SemaphoreType.DMA,
        ],
    )
    def agg(nf_hbm, ef_hbm, src_hbm, dst_hbm, sh_out, ed_out,
            accA, accE, accD,
            sixA, sixB, dix0, dix1, dix2, dix3, dx0, dx1, dx2, dx3,
            gbufA, gbufB, ebtA, ebtB, ebufA, ebufB,
            ones_v, zA, zE, semGA, semGB, semSA, semSB, semDA, semDB,
            semIA, semIB):
        cid = lax.axis_index("c")
        sid = lax.axis_index("s")
        zvec = jnp.zeros((L,), jnp.float32)
        e0 = jnp.where(lax.iota(jnp.int32, L) == 0, 1.0, 0.0)

        def fill_zA(r, _):
            for j in range(DH // L):
                zA[r, pl.ds(j * L, L)] = zvec
            return 0
        lax.fori_loop(0, RI, fill_zA, 0)

        def fill_zE(r, _):
            for j in range(DE // L):
                zE[r, pl.ds(j * L, L)] = zvec
            return 0
        lax.fori_loop(0, RI, fill_zE, 0)

        def fill_ones(r, _):
            ones_v[r, pl.ds(0, L)] = e0
            return 0
        lax.fori_loop(0, C, fill_ones, 0)

        def init_body(k, _):
            c = sid + k * NS
            @pl.when(c < KI)
            def _():
                row = c * RI
                pltpu.sync_copy(zA, accA.at[pl.ds(row, RI)])
                pltpu.sync_copy(zE, accE.at[pl.ds(row, RI)])
                pltpu.sync_copy(zE, accD.at[pl.ds(row, RI)])
            return 0
        lax.fori_loop(0, KI_PT, init_body, 0)
        plsc.subcore_barrier()

        t1 = sid * n1
        t2 = cid * (NS * n2) + sid * n2
        nf_c = nf_hbm.at[cid]

        dixs = (dix0, dix1, dix2, dix3)
        sixs = (sixA, sixB)
        gbufs = (gbufA, gbufB)
        semGs, semSs, semIs = (semGA, semGB), (semSA, semSB), (semIA, semIB)

        def p1_fire_i(b, six, dix, semI):
            for j in range(K):
                ch = t1 + b * K + j
                pltpu.async_copy(src_hbm.at[pl.ds(ch * C1, C1)], six.at[j, 0],
                                 semI)
                pltpu.async_copy(dst_hbm.at[pl.ds(ch * C1, C1)], dix.at[j, 0],
                                 semI)

        def p1_drain_i(semI):
            for j in range(K):
                pltpu.make_async_copy(src_hbm.at[pl.ds(0, C1)], sixA.at[0, 0],
                                      semI).wait()
                pltpu.make_async_copy(dst_hbm.at[pl.ds(0, C1)], dix0.at[0, 0],
                                      semI).wait()

        def p1_fire_g(six, gb, semG):
            for j in range(K):
                pltpu.async_copy(nf_c.at[six.at[j, 0]], gb.at[j], semG)

        def p1_drain_g(semG):
            for j in range(K):
                pltpu.make_async_copy(nf_c.at[sixA.at[0, 0]], gbufA.at[0],
                                      semG).wait()

        def p1_fire_s(gb, dix, semS):
            for j in range(K):
                pltpu.async_copy(gb.at[j], accA.at[dix.at[j, 0]], semS, add=True)

        def p1_drain_s(semS):
            for j in range(K):
                pltpu.make_async_copy(gbufA.at[0], accA.at[dix0.at[0, 0]],
                                      semS).wait()

        def p1_step(b, c, first=False, last=False):
            P = c % 2
            if not last:
                p1_drain_i(semIs[1 - P])
            if not first:
                p1_drain_s(semSs[1 - P])
            if not last:
                p1_fire_g(sixs[1 - P], gbufs[1 - P], semGs[1 - P])
            p1_drain_g(semGs[P])
            if not last:
                def _fire_next_idx():
                    p1_fire_i(b + 2, sixs[P], dixs[(c + 2) % 4], semIs[P])
                if isinstance(b, int):
                    if b + 2 < NB1:
                        _fire_next_idx()
                else:
                    pl.when(b + 2 < NB1)(_fire_next_idx)
            p1_fire_s(gbufs[P], dixs[c % 4], semSs[P])

        p1_fire_i(0, sixA, dix0, semIA)
        p1_fire_i(1, sixB, dix1, semIB)
        p1_drain_i(semIA)
        p1_fire_g(sixA, gbufA, semGA)
        p1_step(0, 0, first=True)

        def p1_body(q, _):
            for c in (1, 2, 3, 4):
                p1_step(4 * q + c, c)
            return 0
        lax.fori_loop(0, (NB1 - 2) // 4, p1_body, 0)

        for b in range(((NB1 - 2) // 4) * 4 + 1, NB1 - 1):
            p1_step(b, b % 4)
        p1_step(NB1 - 1, (NB1 - 1) % 4, last=True)
        p1_drain_s(semSs[(NB1 - 1) % 2])

        ebufs = (ebufA, ebufB)
        ebts = (ebtA, ebtB)
        dxs = (dx0, dx1, dx2, dx3)
        semDs = (semDA, semDB)
        lane = lax.iota(jnp.int32, L)

        def p2_fire_i(b, dix, semI):
            for j in range(K):
                ch = t2 + b * K + j
                pltpu.async_copy(dst_hbm.at[pl.ds(ch * C, C)], dix.at[j, 0], semI)

        def p2_drain_i(semI):
            for j in range(K):
                pltpu.make_async_copy(dst_hbm.at[pl.ds(0, C)], dx0.at[0, 0],
                                      semI).wait()

        RW = C * DE // 128

        def p2_fire_l(b, ebt, semG):
            for j in range(K):
                ch = t2 + b * K + j
                pltpu.async_copy(ef_hbm.at[pl.ds(ch * RW, RW)], ebt.at[j], semG)

        def p2_drain_l(semG):
            for j in range(K):
                pltpu.make_async_copy(ef_hbm.at[pl.ds(0, RW)], ebtA.at[0],
                                      semG).wait()

        def p2_repack(ebt, eb):
            for j in range(K):
                for r in range(C):
                    v = ebt[j, r * DE // 128, pl.ds((r * DE) % 128, DE)]
                    eb[j, r, pl.ds(0, DE)] = v

        def p2_fire_s(eb, dix, semS, semD):
            for j in range(K):
                pltpu.async_copy(eb.at[j], accE.at[dix.at[j, 0]], semS, add=True)
                pltpu.async_copy(ones_v, accD.at[dix.at[j, 0]], semD, add=True)

        def p2_drain_s(semS, semD):
            for j in range(K):
                pltpu.make_async_copy(ebufA.at[0], accE.at[dx0.at[0, 0]],
                                      semS).wait()
                pltpu.make_async_copy(ones_v, accD.at[dx0.at[0, 0]],
                                      semD).wait()

        def p2_step(b, c, first=False, last=False):
            P = c % 2
            if not first:
                p2_drain_s(semSs[1 - P], semDs[1 - P])
            if not last:
                p2_fire_l(b + 1, ebts[1 - P], semGs[1 - P])
            p2_drain_l(semGs[P])
            p2_drain_i(semIs[P])
            if not last:
                def _fire_next_idx():
                    p2_fire_i(b + 2, dxs[(c + 2) % 4], semIs[P])
                if isinstance(b, int):
                    if b + 2 < NB2:
                        _fire_next_idx()
                else:
                    pl.when(b + 2 < NB2)(_fire_next_idx)
            p2_repack(ebts[P], ebufs[P])
            p2_fire_s(ebufs[P], dxs[c % 4], semSs[P], semDs[P])

        p2_fire_i(0, dx0, semIA)
        p2_fire_i(1, dx1, semIB)
        p2_fire_l(0, ebtA, semGA)
        p2_step(0, 0, first=True)

        def p2_body(q, _):
            for c in (1, 2, 3, 4):
                p2_step(4 * q + c, c)
            return 0
        lax.fori_loop(0, (NB2 - 2) // 4, p2_body, 0)

        for b in range(((NB2 - 2) // 4) * 4 + 1, NB2 - 1):
            p2_step(b, b % 4)
        p2_step(NB2 - 1, (NB2 - 1) % 4, last=True)
        p2_drain_s(semSs[(NB2 - 1) % 2], semDs[(NB2 - 1) % 2])

        plsc.subcore_barrier()

        def wb_body(k, _):
            c = sid + k * NS
            @pl.when(c < KI)
            def _():
                row = c * RI
                pltpu.sync_copy(accA.at[pl.ds(row, RI)],
                                sh_out.at[cid, pl.ds(row, RI)])
                pltpu.sync_copy(accE.at[pl.ds(row, RI)],
                                ed_out.at[cid, pl.ds(row, RI), pl.ds(0, DE)])
                pltpu.sync_copy(accD.at[pl.ds(row, RI)],
                                ed_out.at[cid, pl.ds(row, RI), pl.ds(DE, DE)])
            return 0
        lax.fori_loop(0, KI_PT, wb_body, 0)

    return agg(nf_split, ef_r, src, dst)


def _tc_combine_body(DE, x_ref, sh0_ref, sh1_ref, ed0_ref, ed1_ref,
                     wh0_ref, wh1_ref, we_ref, bm_ref,
                     wa1_ref, wa2_ref, ba_ref, out_ref):
    ed = ed0_ref[0] + ed1_ref[0]
    deg = ed[:, DE:DE + 1]
    se = ed[:, :DE]
    summed = (
        jnp.dot(sh0_ref[0], wh0_ref[...], preferred_element_type=jnp.float32)
        + jnp.dot(sh1_ref[0], wh1_ref[...], preferred_element_type=jnp.float32)
        + jnp.dot(se, we_ref[...], preferred_element_type=jnp.float32)
        + deg * bm_ref[...]
    )
    h_neigh = summed / jnp.maximum(deg, 1.0)
    pre = (
        jnp.dot(x_ref[:, 0, :], wa1_ref[...], preferred_element_type=jnp.float32)
        + jnp.dot(h_neigh, wa2_ref[...], preferred_element_type=jnp.float32)
        + ba_ref[...]
    )
    out_ref[:, 0, :] = jnp.maximum(pre, 0.0)


def _tc_combine(nfeats, sh, ed, W_msg, b_msg, W_apply, b_apply, N, DH, DE, DO):
    d_in = 2 * DH
    wh0 = W_msg[:, :DH].T
    wh1 = W_msg[:, DH:d_in].T
    we = W_msg[:, d_in:].T
    wa1 = W_apply[:, :d_in].T
    wa2 = W_apply[:, d_in:].T
    bm = b_msg.reshape(1, DO)
    ba = b_apply.reshape(1, DO)

    R = 2000
    grid = (N + R - 1) // R
    half = lambda c, w: pl.BlockSpec((1, R, w), lambda i, c=c: (c, i, 0))
    full_spec = lambda a: pl.BlockSpec(a.shape, lambda i: (0,) * a.ndim)
    return pl.pallas_call(
        functools.partial(_tc_combine_body, DE),
        grid=(grid,),
        in_specs=[
            pl.BlockSpec((R, 1, d_in), lambda i: (i, 0, 0)),
            half(0, DH), half(1, DH), half(0, 2 * DE), half(1, 2 * DE),
            full_spec(wh0), full_spec(wh1), full_spec(we), full_spec(bm),
            full_spec(wa1), full_spec(wa2), full_spec(ba),
        ],
        out_specs=pl.BlockSpec((R, 1, DO), lambda i: (i, 0, 0)),
        out_shape=jax.ShapeDtypeStruct((N, 1, DO), jnp.float32),
    )(nfeats, sh, sh, ed, ed, wh0, wh1, we, bm, wa1, wa2, ba)


def kernel(nfeats, efeats, edge_index, W_msg, b_msg, W_apply, b_apply):
    N, _, d_in = nfeats.shape
    E = edge_index.shape[1]
    DE = efeats.shape[2]
    DO = W_msg.shape[0]
    DH = d_in // NC

    nf_split = nfeats.reshape(N, NC, DH).transpose(1, 0, 2)

    ef_r = efeats.reshape(E * DE // 128, 128)
    sh, ed = _sc_aggregate(nf_split, ef_r, edge_index[0], edge_index[1],
                           N, E, DH, DE)

    return _tc_combine(nfeats, sh, ed, W_msg, b_msg, W_apply, b_apply,
                       N, DH, DE, DO)


# speedup vs baseline: 1.3537x; 1.3537x over previous
import functools

import jax
import jax.numpy as jnp
from jax import lax
from jax.experimental import pallas as pl
from jax.experimental.pallas import tpu as pltpu
from jax.experimental.pallas import tpu_sc as plsc

NC = 2
NS = 16
L = 16
C = 80
RI = 16
K = 5

_MESH = plsc.VectorSubcoreMesh(core_axis_name="c", subcore_axis_name="s")
_PARAMS = pltpu.CompilerParams(use_tc_tiling_on_sc=False)


def _sc_gather_sum(nf_split, src, dst, N, E, DH):
    per_tile = E // NS
    n1 = per_tile // C
    NB1 = n1 // K
    assert per_tile % C == 0 and n1 % K == 0 and N % RI == 0
    assert NB1 % 2 == 0 and NB1 >= 6
    KI = N // RI
    KI_PT = (KI + NS - 1) // NS

    @functools.partial(
        pl.kernel,
        out_type=jax.ShapeDtypeStruct((NC, N, DH), jnp.float32),
        mesh=_MESH,
        compiler_params=_PARAMS,
        scratch_types=[
            pltpu.VMEM_SHARED((N, DH), jnp.float32),
            pltpu.VMEM((K, 1, C), jnp.int32),
            pltpu.VMEM((K, 1, C), jnp.int32),
            pltpu.VMEM((K, 1, C), jnp.int32),
            pltpu.VMEM((K, 1, C), jnp.int32),
            pltpu.VMEM((K, 1, C), jnp.int32),
            pltpu.VMEM((K, 1, C), jnp.int32),
            pltpu.VMEM((K, C, DH), jnp.float32),
            pltpu.VMEM((K, C, DH), jnp.float32),
            pltpu.VMEM((RI, DH), jnp.float32),
            pltpu.SemaphoreType.DMA,
            pltpu.SemaphoreType.DMA,
            pltpu.SemaphoreType.DMA,
            pltpu.SemaphoreType.DMA,
            pltpu.SemaphoreType.DMA,
            pltpu.SemaphoreType.DMA,
        ],
    )
    def agg(nf_hbm, src_hbm, dst_hbm, sh_out,
            accA, sixA, sixB, dix0, dix1, dix2, dix3, gbufA, gbufB, zA,
            semGA, semGB, semSA, semSB, semIA, semIB):
        cid = lax.axis_index("c")
        sid = lax.axis_index("s")
        zvec = jnp.zeros((L,), jnp.float32)

        def fill_zA(r, _):
            for j in range(DH // L):
                zA[r, pl.ds(j * L, L)] = zvec
            return 0
        lax.fori_loop(0, RI, fill_zA, 0)

        def init_body(k, _):
            c = sid + k * NS
            @pl.when(c < KI)
            def _():
                pltpu.sync_copy(zA, accA.at[pl.ds(c * RI, RI)])
            return 0
        lax.fori_loop(0, KI_PT, init_body, 0)
        plsc.subcore_barrier()

        t1 = sid * n1
        nf_c = nf_hbm.at[cid]

        dixs = (dix0, dix1, dix2, dix3)
        sixs = (sixA, sixB)
        gbufs = (gbufA, gbufB)
        semGs, semSs, semIs = (semGA, semGB), (semSA, semSB), (semIA, semIB)

        def p1_fire_i(b, six, dix, semI):
            for j in range(K):
                ch = t1 + b * K + j
                pltpu.async_copy(src_hbm.at[pl.ds(ch * C, C)], six.at[j, 0], semI)
                pltpu.async_copy(dst_hbm.at[pl.ds(ch * C, C)], dix.at[j, 0], semI)

        def p1_drain_i(semI):
            for j in range(K):
                pltpu.make_async_copy(src_hbm.at[pl.ds(0, C)], sixA.at[0, 0],
                                      semI).wait()
                pltpu.make_async_copy(dst_hbm.at[pl.ds(0, C)], dix0.at[0, 0],
                                      semI).wait()

        def p1_fire_g(six, gb, semG):
            for j in range(K):
                pltpu.async_copy(nf_c.at[six.at[j, 0]], gb.at[j], semG)

        def p1_drain_g(semG):
            for j in range(K):
                pltpu.make_async_copy(nf_c.at[sixA.at[0, 0]], gbufA.at[0],
                                      semG).wait()

        def p1_fire_s(gb, dix, semS):
            for j in range(K):
                pltpu.async_copy(gb.at[j], accA.at[dix.at[j, 0]], semS, add=True)

        def p1_drain_s(semS):
            for j in range(K):
                pltpu.make_async_copy(gbufA.at[0], accA.at[dix0.at[0, 0]],
                                      semS).wait()

        def p1_step(b, c, first=False, last=False):
            P = c % 2
            if not last:
                p1_drain_i(semIs[1 - P])
            if not first:
                p1_drain_s(semSs[1 - P])
            if not last:
                p1_fire_g(sixs[1 - P], gbufs[1 - P], semGs[1 - P])
            p1_drain_g(semGs[P])
            if not last:
                def _fire_next_idx():
                    p1_fire_i(b + 2, sixs[P], dixs[(c + 2) % 4], semIs[P])
                if isinstance(b, int):
                    if b + 2 < NB1:
                        _fire_next_idx()
                else:
                    pl.when(b + 2 < NB1)(_fire_next_idx)
            p1_fire_s(gbufs[P], dixs[c % 4], semSs[P])

        p1_fire_i(0, sixA, dix0, semIA)
        p1_fire_i(1, sixB, dix1, semIB)
        p1_drain_i(semIA)
        p1_fire_g(sixA, gbufA, semGA)
        p1_step(0, 0, first=True)

        def p1_body(q, _):
            for c in (1, 2, 3, 4):
                p1_step(4 * q + c, c)
            return 0
        lax.fori_loop(0, (NB1 - 2) // 4, p1_body, 0)

        for b in range(((NB1 - 2) // 4) * 4 + 1, NB1 - 1):
            p1_step(b, b % 4)
        p1_step(NB1 - 1, (NB1 - 1) % 4, last=True)
        p1_drain_s(semSs[(NB1 - 1) % 2])

        plsc.subcore_barrier()

        def wb_body(k, _):
            c = sid + k * NS
            @pl.when(c < KI)
            def _():
                pltpu.sync_copy(accA.at[pl.ds(c * RI, RI)],
                                sh_out.at[cid, pl.ds(c * RI, RI)])
            return 0
        lax.fori_loop(0, KI_PT, wb_body, 0)

    return agg(nf_split, src, dst)


def _sc_edge_sum(ef3, dst, N, E, DE):
    per_tile = E // (NC * NS)
    n2 = per_tile // C
    NB2 = n2 // K
    assert per_tile % C == 0 and n2 % K == 0 and N % RI == 0
    assert NB2 % 2 == 1 and NB2 >= 3
    KI = N // RI
    KI_PT = (KI + NS - 1) // NS

    @functools.partial(
        pl.kernel,
        out_type=jax.ShapeDtypeStruct((NC, N, 2 * DE), jnp.float32),
        mesh=_MESH,
        compiler_params=_PARAMS,
        scratch_types=[
            pltpu.VMEM_SHARED((N, DE), jnp.float32),
            pltpu.VMEM_SHARED((N, DE), jnp.float32),
            pltpu.VMEM((K, 1, C), jnp.int32),
            pltpu.VMEM((K, 1, C), jnp.int32),
            pltpu.VMEM((K, 1, C), jnp.int32),
            pltpu.VMEM((K, 1, C), jnp.int32),
            pltpu.VMEM((K, C, DE), jnp.float32),
            pltpu.VMEM((K, C, DE), jnp.float32),
            pltpu.VMEM((C, DE), jnp.float32),
            pltpu.VMEM((RI, DE), jnp.float32),
            pltpu.SemaphoreType.DMA,
            pltpu.SemaphoreType.DMA,
            pltpu.SemaphoreType.DMA,
            pltpu.SemaphoreType.DMA,
            pltpu.SemaphoreType.DMA,
            pltpu.SemaphoreType.DMA,
            pltpu.SemaphoreType.DMA,
            pltpu.SemaphoreType.DMA,
        ],
    )
    def agg(ef_hbm, dst_hbm, ed_out,
            accE, accD, dix0, dix1, dix2, dix3, ebufA, ebufB, ones_v, zE,
            semGA, semGB, semSA, semSB, semDA, semDB, semIA, semIB):
        cid = lax.axis_index("c")
        sid = lax.axis_index("s")
        zvec = jnp.zeros((L,), jnp.float32)
        e0 = jnp.where(lax.iota(jnp.int32, L) == 0, 1.0, 0.0)

        def fill_zE(r, _):
            for j in range(DE // L):
                zE[r, pl.ds(j * L, L)] = zvec
            return 0
        lax.fori_loop(0, RI, fill_zE, 0)

        def fill_ones(r, _):
            ones_v[r, pl.ds(0, L)] = e0
            return 0
        lax.fori_loop(0, C, fill_ones, 0)

        def init_body(k, _):
            c = sid + k * NS
            @pl.when(c < KI)
            def _():
                pltpu.sync_copy(zE, accE.at[pl.ds(c * RI, RI)])
                pltpu.sync_copy(zE, accD.at[pl.ds(c * RI, RI)])
            return 0
        lax.fori_loop(0, KI_PT, init_body, 0)
        plsc.subcore_barrier()

        t2 = cid * (NS * n2) + sid * n2
        dixs = (dix0, dix1, dix2, dix3)
        ebufs = (ebufA, ebufB)
        semGs, semSs = (semGA, semGB), (semSA, semSB)
        semDs, semIs = (semDA, semDB), (semIA, semIB)

        def p2_fire_i(b, dix, semI):
            for j in range(K):
                ch = t2 + b * K + j
                pltpu.async_copy(dst_hbm.at[pl.ds(ch * C, C)], dix.at[j, 0], semI)

        def p2_drain_i(semI):
            for j in range(K):
                pltpu.make_async_copy(dst_hbm.at[pl.ds(0, C)], dix0.at[0, 0],
                                      semI).wait()

        def p2_fire_l(b, eb, semG):
            for j in range(K):
                pltpu.async_copy(ef_hbm.at[pl.ds((t2 + b * K + j) * C, C), 0],
                                 eb.at[j], semG)

        def p2_drain_l(semG):
            for j in range(K):
                pltpu.make_async_copy(ef_hbm.at[pl.ds(0, C), 0], ebufA.at[0],
                                      semG).wait()

        def p2_fire_s(eb, dix, semS, semD):
            for j in range(K):
                pltpu.async_copy(eb.at[j], accE.at[dix.at[j, 0]], semS, add=True)
                pltpu.async_copy(ones_v, accD.at[dix.at[j, 0]], semD, add=True)

        def p2_drain_s(semS, semD):
            for j in range(K):
                pltpu.make_async_copy(ebufA.at[0], accE.at[dix0.at[0, 0]],
                                      semS).wait()
                pltpu.make_async_copy(ones_v, accD.at[dix0.at[0, 0]],
                                      semD).wait()

        def p2_step(b, c, first=False, last=False):
            P = c % 2
            if not first:
                p2_drain_s(semSs[1 - P], semDs[1 - P])
            if not last:
                p2_fire_l(b + 1, ebufs[1 - P], semGs[1 - P])
            p2_drain_l(semGs[P])
            p2_drain_i(semIs[P])
            if not last:
                def _fire_next_idx():
                    p2_fire_i(b + 2, dixs[(c + 2) % 4], semIs[P])
                if isinstance(b, int):
                    if b + 2 < NB2:
                        _fire_next_idx()
                else:
                    pl.when(b + 2 < NB2)(_fire_next_idx)
            p2_fire_s(ebufs[P], dixs[c % 4], semSs[P], semDs[P])

        p2_fire_i(0, dix0, semIA)
        p2_fire_i(1, dix1, semIB)
        p2_fire_l(0, ebufA, semGA)
        p2_step(0, 0, first=True)

        def p2_body(q, _):
            for c in (1, 2, 3, 4):
                p2_step(4 * q + c, c)
            return 0
        lax.fori_loop(0, (NB2 - 2) // 4, p2_body, 0)

        for b in range(((NB2 - 2) // 4) * 4 + 1, NB2 - 1):
            p2_step(b, b % 4)
        p2_step(NB2 - 1, (NB2 - 1) % 4, last=True)
        p2_drain_s(semSs[(NB2 - 1) % 2], semDs[(NB2 - 1) % 2])

        plsc.subcore_barrier()

        def wb_body(k, _):
            c = sid + k * NS
            @pl.when(c < KI)
            def _():
                row = c * RI
                pltpu.sync_copy(accE.at[pl.ds(row, RI)],
                                ed_out.at[cid, pl.ds(row, RI), pl.ds(0, DE)])
                pltpu.sync_copy(accD.at[pl.ds(row, RI)],
                                ed_out.at[cid, pl.ds(row, RI), pl.ds(DE, DE)])
            return 0
        lax.fori_loop(0, KI_PT, wb_body, 0)

    return agg(ef3, dst)


def _tc_combine_body(DE, x_ref, sh0_ref, sh1_ref, ed0_ref, ed1_ref,
                     wh0_ref, wh1_ref, we_ref, bm_ref,
                     wa1_ref, wa2_ref, ba_ref, out_ref):
    ed = ed0_ref[0] + ed1_ref[0]
    deg = ed[:, DE:DE + 1]
    se = ed[:, :DE]
    summed = (
        jnp.dot(sh0_ref[0], wh0_ref[...], preferred_element_type=jnp.float32)
        + jnp.dot(sh1_ref[0], wh1_ref[...], preferred_element_type=jnp.float32)
        + jnp.dot(se, we_ref[...], preferred_element_type=jnp.float32)
        + deg * bm_ref[...]
    )
    h_neigh = summed / jnp.maximum(deg, 1.0)
    pre = (
        jnp.dot(x_ref[:, 0, :], wa1_ref[...], preferred_element_type=jnp.float32)
        + jnp.dot(h_neigh, wa2_ref[...], preferred_element_type=jnp.float32)
        + ba_ref[...]
    )
    out_ref[:, 0, :] = jnp.maximum(pre, 0.0)


def _tc_combine(nfeats, sh, ed, W_msg, b_msg, W_apply, b_apply, N, DH, DE, DO):
    d_in = 2 * DH
    wh0 = W_msg[:, :DH].T
    wh1 = W_msg[:, DH:d_in].T
    we = W_msg[:, d_in:].T
    wa1 = W_apply[:, :d_in].T
    wa2 = W_apply[:, d_in:].T
    bm = b_msg.reshape(1, DO)
    ba = b_apply.reshape(1, DO)

    R = 2000
    grid = (N + R - 1) // R
    half = lambda c, w: pl.BlockSpec((1, R, w), lambda i, c=c: (c, i, 0))
    full_spec = lambda a: pl.BlockSpec(a.shape, lambda i: (0,) * a.ndim)
    return pl.pallas_call(
        functools.partial(_tc_combine_body, DE),
        grid=(grid,),
        in_specs=[
            pl.BlockSpec((R, 1, d_in), lambda i: (i, 0, 0)),
            half(0, DH), half(1, DH), half(0, 2 * DE), half(1, 2 * DE),
            full_spec(wh0), full_spec(wh1), full_spec(we), full_spec(bm),
            full_spec(wa1), full_spec(wa2), full_spec(ba),
        ],
        out_specs=pl.BlockSpec((R, 1, DO), lambda i: (i, 0, 0)),
        out_shape=jax.ShapeDtypeStruct((N, 1, DO), jnp.float32),
    )(nfeats, sh, sh, ed, ed, wh0, wh1, we, bm, wa1, wa2, ba)


def kernel(nfeats, efeats, edge_index, W_msg, b_msg, W_apply, b_apply):
    N, _, d_in = nfeats.shape
    E = edge_index.shape[1]
    DE = efeats.shape[2]
    DO = W_msg.shape[0]
    DH = d_in // NC

    nf_split = nfeats.reshape(N, NC, DH).transpose(1, 0, 2)
    src = edge_index[0]
    dst = edge_index[1]

    sh = _sc_gather_sum(nf_split, src, dst, N, E, DH)
    ed = _sc_edge_sum(efeats, dst, N, E, DE)

    return _tc_combine(nfeats, sh, ed, W_msg, b_msg, W_apply, b_apply,
                       N, DH, DE, DO)


# speedup vs baseline: 1.3781x; 1.0181x over previous
import functools

import jax
import jax.numpy as jnp
from jax import lax
from jax.experimental import pallas as pl
from jax.experimental.pallas import tpu as pltpu
from jax.experimental.pallas import tpu_sc as plsc

NC = 2
NS = 16
L = 16
C = 80
RI = 16
K = 5

_MESH = plsc.VectorSubcoreMesh(core_axis_name="c", subcore_axis_name="s")
_PARAMS = pltpu.CompilerParams(use_tc_tiling_on_sc=False)


def _sc_gather_sum(nf_split, edge_index, N, E, DH):
    per_tile = E // NS
    n1 = per_tile // C
    NB1 = n1 // K
    assert per_tile % C == 0 and n1 % K == 0 and N % RI == 0
    assert NB1 % 2 == 0 and NB1 >= 6
    KI = N // RI
    KI_PT = (KI + NS - 1) // NS

    @functools.partial(
        pl.kernel,
        out_type=jax.ShapeDtypeStruct((NC, N, DH), jnp.float32),
        mesh=_MESH,
        compiler_params=_PARAMS,
        scratch_types=[
            pltpu.VMEM_SHARED((N, DH), jnp.float32),
            pltpu.VMEM((K, 1, C), jnp.int32),
            pltpu.VMEM((K, 1, C), jnp.int32),
            pltpu.VMEM((K, 1, C), jnp.int32),
            pltpu.VMEM((K, 1, C), jnp.int32),
            pltpu.VMEM((K, 1, C), jnp.int32),
            pltpu.VMEM((K, 1, C), jnp.int32),
            pltpu.VMEM((K, C, DH), jnp.float32),
            pltpu.VMEM((K, C, DH), jnp.float32),
            pltpu.VMEM((RI, DH), jnp.float32),
            pltpu.SemaphoreType.DMA,
            pltpu.SemaphoreType.DMA,
            pltpu.SemaphoreType.DMA,
            pltpu.SemaphoreType.DMA,
            pltpu.SemaphoreType.DMA,
            pltpu.SemaphoreType.DMA,
        ],
    )
    def agg(nf_hbm, ei_hbm, sh_out,
            accA, sixA, sixB, dix0, dix1, dix2, dix3, gbufA, gbufB, zA,
            semGA, semGB, semSA, semSB, semIA, semIB):
        cid = lax.axis_index("c")
        sid = lax.axis_index("s")
        zvec = jnp.zeros((L,), jnp.float32)

        def fill_zA(r, _):
            for j in range(DH // L):
                zA[r, pl.ds(j * L, L)] = zvec
            return 0
        lax.fori_loop(0, RI, fill_zA, 0)

        def init_body(k, _):
            c = sid + k * NS
            @pl.when(c < KI)
            def _():
                pltpu.sync_copy(zA, accA.at[pl.ds(c * RI, RI)])
            return 0
        lax.fori_loop(0, KI_PT, init_body, 0)
        plsc.subcore_barrier()

        t1 = sid * n1
        nf_c = nf_hbm.at[cid]

        dixs = (dix0, dix1, dix2, dix3)
        sixs = (sixA, sixB)
        gbufs = (gbufA, gbufB)
        semGs, semSs, semIs = (semGA, semGB), (semSA, semSB), (semIA, semIB)

        def p1_fire_i(b, six, dix, semI):
            for j in range(K):
                ch = t1 + b * K + j
                pltpu.async_copy(ei_hbm.at[0, pl.ds(ch * C, C)], six.at[j, 0],
                                 semI)
                pltpu.async_copy(ei_hbm.at[1, pl.ds(ch * C, C)], dix.at[j, 0],
                                 semI)

        def p1_drain_i(semI):
            for j in range(K):
                pltpu.make_async_copy(ei_hbm.at[0, pl.ds(0, C)], sixA.at[0, 0],
                                      semI).wait()
                pltpu.make_async_copy(ei_hbm.at[1, pl.ds(0, C)], dix0.at[0, 0],
                                      semI).wait()

        def p1_fire_g(six, gb, semG):
            for j in range(K):
                pltpu.async_copy(nf_c.at[six.at[j, 0]], gb.at[j], semG)

        def p1_drain_g(semG):
            for j in range(K):
                pltpu.make_async_copy(nf_c.at[sixA.at[0, 0]], gbufA.at[0],
                                      semG).wait()

        def p1_fire_s(gb, dix, semS):
            for j in range(K):
                pltpu.async_copy(gb.at[j], accA.at[dix.at[j, 0]], semS, add=True)

        def p1_drain_s(semS):
            for j in range(K):
                pltpu.make_async_copy(gbufA.at[0], accA.at[dix0.at[0, 0]],
                                      semS).wait()

        def p1_step(b, c, first=False, last=False):
            P = c % 2
            if not last:
                p1_drain_i(semIs[1 - P])
            if not first:
                p1_drain_s(semSs[1 - P])
            if not last:
                p1_fire_g(sixs[1 - P], gbufs[1 - P], semGs[1 - P])
            p1_drain_g(semGs[P])
            if not last:
                def _fire_next_idx():
                    p1_fire_i(b + 2, sixs[P], dixs[(c + 2) % 4], semIs[P])
                if isinstance(b, int):
                    if b + 2 < NB1:
                        _fire_next_idx()
                else:
                    pl.when(b + 2 < NB1)(_fire_next_idx)
            p1_fire_s(gbufs[P], dixs[c % 4], semSs[P])

        p1_fire_i(0, sixA, dix0, semIA)
        p1_fire_i(1, sixB, dix1, semIB)
        p1_drain_i(semIA)
        p1_fire_g(sixA, gbufA, semGA)
        p1_step(0, 0, first=True)

        def p1_body(q, _):
            for c in (1, 2, 3, 4):
                p1_step(4 * q + c, c)
            return 0
        lax.fori_loop(0, (NB1 - 2) // 4, p1_body, 0)

        for b in range(((NB1 - 2) // 4) * 4 + 1, NB1 - 1):
            p1_step(b, b % 4)
        p1_step(NB1 - 1, (NB1 - 1) % 4, last=True)
        p1_drain_s(semSs[(NB1 - 1) % 2])

        plsc.subcore_barrier()

        def wb_body(k, _):
            c = sid + k * NS
            @pl.when(c < KI)
            def _():
                pltpu.sync_copy(accA.at[pl.ds(c * RI, RI)],
                                sh_out.at[cid, pl.ds(c * RI, RI)])
            return 0
        lax.fori_loop(0, KI_PT, wb_body, 0)

    return agg(nf_split, edge_index)


def _sc_edge_sum(ef3, edge_index, N, E, DE):
    per_tile = E // (NC * NS)
    n2 = per_tile // C
    NB2 = n2 // K
    assert per_tile % C == 0 and n2 % K == 0 and N % RI == 0
    assert NB2 % 2 == 1 and NB2 >= 3
    KI = N // RI
    KI_PT = (KI + NS - 1) // NS

    @functools.partial(
        pl.kernel,
        out_type=jax.ShapeDtypeStruct((NC, N, 2 * DE), jnp.float32),
        mesh=_MESH,
        compiler_params=_PARAMS,
        scratch_types=[
            pltpu.VMEM_SHARED((N, DE), jnp.float32),
            pltpu.VMEM_SHARED((N, DE), jnp.float32),
            pltpu.VMEM((K, 1, C), jnp.int32),
            pltpu.VMEM((K, 1, C), jnp.int32),
            pltpu.VMEM((K, 1, C), jnp.int32),
            pltpu.VMEM((K, 1, C), jnp.int32),
            pltpu.VMEM((K, C, DE), jnp.float32),
            pltpu.VMEM((K, C, DE), jnp.float32),
            pltpu.VMEM((C, DE), jnp.float32),
            pltpu.VMEM((RI, DE), jnp.float32),
            pltpu.SemaphoreType.DMA,
            pltpu.SemaphoreType.DMA,
            pltpu.SemaphoreType.DMA,
            pltpu.SemaphoreType.DMA,
            pltpu.SemaphoreType.DMA,
            pltpu.SemaphoreType.DMA,
            pltpu.SemaphoreType.DMA,
            pltpu.SemaphoreType.DMA,
        ],
    )
    def agg(ef_hbm, ei_hbm, ed_out,
            accE, accD, dix0, dix1, dix2, dix3, ebufA, ebufB, ones_v, zE,
            semGA, semGB, semSA, semSB, semDA, semDB, semIA, semIB):
        cid = lax.axis_index("c")
        sid = lax.axis_index("s")
        zvec = jnp.zeros((L,), jnp.float32)
        e0 = jnp.where(lax.iota(jnp.int32, L) == 0, 1.0, 0.0)

        def fill_zE(r, _):
            for j in range(DE // L):
                zE[r, pl.ds(j * L, L)] = zvec
            return 0
        lax.fori_loop(0, RI, fill_zE, 0)

        def fill_ones(r, _):
            ones_v[r, pl.ds(0, L)] = e0
            return 0
        lax.fori_loop(0, C, fill_ones, 0)

        def init_body(k, _):
            c = sid + k * NS
            @pl.when(c < KI)
            def _():
                pltpu.sync_copy(zE, accE.at[pl.ds(c * RI, RI)])
                pltpu.sync_copy(zE, accD.at[pl.ds(c * RI, RI)])
            return 0
        lax.fori_loop(0, KI_PT, init_body, 0)
        plsc.subcore_barrier()

        t2 = cid * (NS * n2) + sid * n2
        dixs = (dix0, dix1, dix2, dix3)
        ebufs = (ebufA, ebufB)
        semGs, semSs = (semGA, semGB), (semSA, semSB)
        semDs, semIs = (semDA, semDB), (semIA, semIB)

        def p2_fire_i(b, dix, semI):
            for j in range(K):
                ch = t2 + b * K + j
                pltpu.async_copy(ei_hbm.at[1, pl.ds(ch * C, C)], dix.at[j, 0],
                                 semI)

        def p2_drain_i(semI):
            for j in range(K):
                pltpu.make_async_copy(ei_hbm.at[1, pl.ds(0, C)], dix0.at[0, 0],
                                      semI).wait()

        def p2_fire_l(b, eb, semG):
            for j in range(K):
                pltpu.async_copy(ef_hbm.at[pl.ds((t2 + b * K + j) * C, C), 0],
                                 eb.at[j], semG)

        def p2_drain_l(semG):
            for j in range(K):
                pltpu.make_async_copy(ef_hbm.at[pl.ds(0, C), 0], ebufA.at[0],
                                      semG).wait()

        def p2_fire_s(eb, dix, semS, semD):
            for j in range(K):
                pltpu.async_copy(eb.at[j], accE.at[dix.at[j, 0]], semS, add=True)
                pltpu.async_copy(ones_v, accD.at[dix.at[j, 0]], semD, add=True)

        def p2_drain_s(semS, semD):
            for j in range(K):
                pltpu.make_async_copy(ebufA.at[0], accE.at[dix0.at[0, 0]],
                                      semS).wait()
                pltpu.make_async_copy(ones_v, accD.at[dix0.at[0, 0]],
                                      semD).wait()

        def p2_step(b, c, first=False, last=False):
            P = c % 2
            if not first:
                p2_drain_s(semSs[1 - P], semDs[1 - P])
            if not last:
                p2_fire_l(b + 1, ebufs[1 - P], semGs[1 - P])
            p2_drain_l(semGs[P])
            p2_drain_i(semIs[P])
            if not last:
                def _fire_next_idx():
                    p2_fire_i(b + 2, dixs[(c + 2) % 4], semIs[P])
                if isinstance(b, int):
                    if b + 2 < NB2:
                        _fire_next_idx()
                else:
                    pl.when(b + 2 < NB2)(_fire_next_idx)
            p2_fire_s(ebufs[P], dixs[c % 4], semSs[P], semDs[P])

        p2_fire_i(0, dix0, semIA)
        p2_fire_i(1, dix1, semIB)
        p2_fire_l(0, ebufA, semGA)
        p2_step(0, 0, first=True)

        def p2_body(q, _):
            for c in (1, 2, 3, 4):
                p2_step(4 * q + c, c)
            return 0
        lax.fori_loop(0, (NB2 - 2) // 4, p2_body, 0)

        for b in range(((NB2 - 2) // 4) * 4 + 1, NB2 - 1):
            p2_step(b, b % 4)
        p2_step(NB2 - 1, (NB2 - 1) % 4, last=True)
        p2_drain_s(semSs[(NB2 - 1) % 2], semDs[(NB2 - 1) % 2])

        plsc.subcore_barrier()

        def wb_body(k, _):
            c = sid + k * NS
            @pl.when(c < KI)
            def _():
                row = c * RI
                pltpu.sync_copy(accE.at[pl.ds(row, RI)],
                                ed_out.at[cid, pl.ds(row, RI), pl.ds(0, DE)])
                pltpu.sync_copy(accD.at[pl.ds(row, RI)],
                                ed_out.at[cid, pl.ds(row, RI), pl.ds(DE, DE)])
            return 0
        lax.fori_loop(0, KI_PT, wb_body, 0)

    return agg(ef3, edge_index)


def _tc_combine_body(DE, x_ref, sh0_ref, sh1_ref, ed0_ref, ed1_ref,
                     wh0_ref, wh1_ref, we_ref, bm_ref,
                     wa1_ref, wa2_ref, ba_ref, out_ref):
    ed = ed0_ref[0] + ed1_ref[0]
    deg = ed[:, DE:DE + 1]
    se = ed[:, :DE]
    summed = (
        jnp.dot(sh0_ref[0], wh0_ref[...], preferred_element_type=jnp.float32)
        + jnp.dot(sh1_ref[0], wh1_ref[...], preferred_element_type=jnp.float32)
        + jnp.dot(se, we_ref[...], preferred_element_type=jnp.float32)
        + deg * bm_ref[...]
    )
    h_neigh = summed / jnp.maximum(deg, 1.0)
    pre = (
        jnp.dot(x_ref[:, 0, :], wa1_ref[...], preferred_element_type=jnp.float32)
        + jnp.dot(h_neigh, wa2_ref[...], preferred_element_type=jnp.float32)
        + ba_ref[...]
    )
    out_ref[:, 0, :] = jnp.maximum(pre, 0.0)


def _tc_combine(nfeats, sh, ed, W_msg, b_msg, W_apply, b_apply, N, DH, DE, DO):
    d_in = 2 * DH
    wh0 = W_msg[:, :DH].T
    wh1 = W_msg[:, DH:d_in].T
    we = W_msg[:, d_in:].T
    wa1 = W_apply[:, :d_in].T
    wa2 = W_apply[:, d_in:].T
    bm = b_msg.reshape(1, DO)
    ba = b_apply.reshape(1, DO)

    R = 2000
    grid = (N + R - 1) // R
    half = lambda c, w: pl.BlockSpec((1, R, w), lambda i, c=c: (c, i, 0))
    full_spec = lambda a: pl.BlockSpec(a.shape, lambda i: (0,) * a.ndim)
    return pl.pallas_call(
        functools.partial(_tc_combine_body, DE),
        grid=(grid,),
        in_specs=[
            pl.BlockSpec((R, 1, d_in), lambda i: (i, 0, 0)),
            half(0, DH), half(1, DH), half(0, 2 * DE), half(1, 2 * DE),
            full_spec(wh0), full_spec(wh1), full_spec(we), full_spec(bm),
            full_spec(wa1), full_spec(wa2), full_spec(ba),
        ],
        out_specs=pl.BlockSpec((R, 1, DO), lambda i: (i, 0, 0)),
        out_shape=jax.ShapeDtypeStruct((N, 1, DO), jnp.float32),
    )(nfeats, sh, sh, ed, ed, wh0, wh1, we, bm, wa1, wa2, ba)


def kernel(nfeats, efeats, edge_index, W_msg, b_msg, W_apply, b_apply):
    N, _, d_in = nfeats.shape
    E = edge_index.shape[1]
    DE = efeats.shape[2]
    DO = W_msg.shape[0]
    DH = d_in // NC

    nf_split = nfeats.reshape(N, NC, DH).transpose(1, 0, 2)

    sh = _sc_gather_sum(nf_split, edge_index, N, E, DH)
    ed = _sc_edge_sum(efeats, edge_index, N, E, DE)

    return _tc_combine(nfeats, sh, ed, W_msg, b_msg, W_apply, b_apply,
                       N, DH, DE, DO)
